# Initial kernel scaffold; baseline (speedup 1.0000x reference)
import jax
import jax.numpy as jnp
from jax.experimental import pallas as pl


def kernel(x, ner, pos, entity_table):
    raise NotImplementedError("write your pallas kernel here")


# speedup vs baseline: 2.1504x; 2.1504x over previous
import functools

import jax
import jax.numpy as jnp
from jax import lax
from jax.experimental import pallas as pl
from jax.experimental.pallas import tpu as pltpu
from jax.experimental.pallas import tpu_sc as plsc

NC = 2
NS = 16
NW = NC * NS

IDX_GRP = 128


def _sc_gather(table, idx_flat, n_total, emb_dim):
    grps_total = n_total // IDX_GRP
    grps_per_w = grps_total // NW
    rows_per_w = grps_per_w * IDX_GRP
    grps_per_chunk = max(
        g for g in range(1, 13)
        if grps_per_w % g == 0
        and g * IDX_GRP * emb_dim * 4 <= 256 * 1024)
    n_chunks = grps_per_w // grps_per_chunk
    rows_per_chunk = grps_per_chunk * IDX_GRP

    mesh = plsc.VectorSubcoreMesh(
        core_axis_name="c", subcore_axis_name="s", num_cores=NC,
        num_subcores=NS)

    @functools.partial(
        pl.kernel,
        out_type=jax.ShapeDtypeStruct((n_total, emb_dim), jnp.float32),
        mesh=mesh,
        scratch_types=[
            pltpu.VMEM((rows_per_w,), jnp.int32),
            pltpu.VMEM((rows_per_chunk, emb_dim), jnp.float32),
            pltpu.SemaphoreType.DMA,
        ],
        compiler_params=pltpu.CompilerParams(use_tc_tiling_on_sc=False),
    )
    def gather_kernel(table_hbm, idx_hbm, out_hbm, idx_v, rows_v, sem):
        wid = lax.axis_index("s") * NC + lax.axis_index("c")
        row_base = wid * rows_per_w
        pltpu.sync_copy(idx_hbm.at[pl.ds(row_base, rows_per_w)], idx_v)

        def chunk_body(c, _):
            copies = []
            for j in range(grps_per_chunk):
                g = c * grps_per_chunk + j
                copies.append(pltpu.async_copy(
                    table_hbm.at[idx_v.at[pl.ds(g * IDX_GRP, IDX_GRP)]],
                    rows_v.at[pl.ds(j * IDX_GRP, IDX_GRP)],
                    sem,
                ))
            for cp in copies:
                cp.wait()
            pltpu.sync_copy(
                rows_v,
                out_hbm.at[pl.ds(row_base + c * rows_per_chunk,
                                 rows_per_chunk)],
            )
            return ()

        lax.fori_loop(0, n_chunks, chunk_body, (), unroll=False)

    return gather_kernel(table, idx_flat)


def _concat_kernel(x_ref, emb_ref, out_ref):
    out_ref[:, : x_ref.shape[1]] = x_ref[...]
    out_ref[:, x_ref.shape[1]:] = emb_ref[...]


def kernel(x, ner, pos, entity_table):
    B, S, D = x.shape
    V, E = entity_table.shape
    n = B * S

    idx = ner.reshape(n).astype(jnp.int32)
    emb = _sc_gather(entity_table, idx, n, E)

    x2 = x.reshape(n, D)
    ROWS = 2048
    out = pl.pallas_call(
        _concat_kernel,
        grid=(n // ROWS,),
        in_specs=[
            pl.BlockSpec((ROWS, D), lambda i: (i, 0)),
            pl.BlockSpec((ROWS, E), lambda i: (i, 0)),
        ],
        out_specs=pl.BlockSpec((ROWS, D + E), lambda i: (i, 0)),
        out_shape=jax.ShapeDtypeStruct((n, D + E), jnp.float32),
    )(x2, emb)
    return out.reshape(B, S, D + E)
